# baseline (speedup 1.0000x reference)
import jax
import jax.numpy as jnp
from jax import lax
from jax.experimental import pallas as pl
from jax.experimental.pallas import tpu as pltpu
from jax.experimental.pallas import tpu_sc as plsc

B = 4
S = 8192
D = 4096
E = 64

NC = 2
NS = 16
NW = NC * NS
CH = 8
NLANE = 16
UNROLL = 4


def _sc_sum_body(x_hbm, out_hbm, buf, acc, sem0, sem1):
    wid = lax.axis_index("s") * NC + lax.axis_index("c")
    rows_w = (B * S) // NW
    nch = rows_w // CH
    base = wid * rows_w

    def _zero(k, _):
        acc[pl.ds(k * NLANE, NLANE)] = jnp.zeros((NLANE,), jnp.float32)
        return 0

    lax.fori_loop(0, D // NLANE, _zero, 0)

    def _copy(c, slot, sem):
        return pltpu.make_async_copy(
            x_hbm.at[pl.ds(base + c * CH, CH), :], buf.at[slot], sem)

    _copy(0, 0, sem0).start()
    _copy(1, 1, sem1).start()

    def _chunk(c, slot, sem):
        _copy(c, slot, sem).wait()

        def _acc_slices(k, _):
            for u in range(UNROLL):
                o = (k * UNROLL + u) * NLANE
                t0 = buf[slot, 0, pl.ds(o, NLANE)] + buf[slot, 1, pl.ds(o, NLANE)]
                t1 = buf[slot, 2, pl.ds(o, NLANE)] + buf[slot, 3, pl.ds(o, NLANE)]
                t2 = buf[slot, 4, pl.ds(o, NLANE)] + buf[slot, 5, pl.ds(o, NLANE)]
                t3 = buf[slot, 6, pl.ds(o, NLANE)] + buf[slot, 7, pl.ds(o, NLANE)]
                acc[pl.ds(o, NLANE)] += (t0 + t1) + (t2 + t3)
            return 0

        lax.fori_loop(0, D // NLANE // UNROLL, _acc_slices, 0)

        @pl.when(c + 2 < nch)
        def _next():
            _copy(c + 2, slot, sem).start()

    def _pair(i, _):
        c0 = i * 2
        _chunk(c0, 0, sem0)
        _chunk(c0 + 1, 1, sem1)
        return 0

    lax.fori_loop(0, nch // 2, _pair, 0)

    pltpu.sync_copy(acc, out_hbm.at[wid])


def _sc_sum(xf):
    mesh = plsc.VectorSubcoreMesh(core_axis_name="c", subcore_axis_name="s")
    return pl.kernel(
        _sc_sum_body,
        out_type=jax.ShapeDtypeStruct((NW, D), jnp.float32),
        mesh=mesh,
        scratch_types=[
            pltpu.VMEM((2, CH, D), jnp.float32),
            pltpu.VMEM((D,), jnp.float32),
            pltpu.SemaphoreType.DMA,
            pltpu.SemaphoreType.DMA,
        ],
    )(xf)


def _finish_body(p_ref, w_ref, b_ref, o_ref):
    sel = (jax.lax.broadcasted_iota(jnp.int32, (B, NW), 1) // (NW // B)
           == jax.lax.broadcasted_iota(jnp.int32, (B, NW), 0)).astype(jnp.float32)
    xbar = jax.lax.dot_general(
        sel, p_ref[...],
        dimension_numbers=(((1,), (0,)), ((), ())),
        preferred_element_type=jnp.float32,
    ) * (1.0 / S)
    scores = jax.lax.dot_general(
        xbar, w_ref[...],
        dimension_numbers=(((1,), (1,)), ((), ())),
        preferred_element_type=jnp.float32,
    ) + b_ref[...]
    m = jnp.max(scores, axis=1, keepdims=True)
    ex = jnp.exp(scores - m)
    o_ref[...] = ex / jnp.sum(ex, axis=1, keepdims=True)


def _finish(partials, W, b2):
    return pl.pallas_call(
        _finish_body,
        in_specs=[
            pl.BlockSpec((NW, D), lambda: (0, 0)),
            pl.BlockSpec((E, D), lambda: (0, 0)),
            pl.BlockSpec((1, E), lambda: (0, 0)),
        ],
        out_specs=pl.BlockSpec((B, E), lambda: (0, 0)),
        out_shape=jax.ShapeDtypeStruct((B, E), jnp.float32),
    )(partials, W, b2)


def kernel(x, W, b):
    xf = x.reshape(B * S, D)
    partials = _sc_sum(xf)
    return _finish(partials, W, b.reshape(1, E))


# speedup vs baseline: 2.4285x; 2.4285x over previous
import jax
import jax.numpy as jnp
from jax import lax
from jax.experimental import pallas as pl
from jax.experimental.pallas import tpu as pltpu
from jax.experimental.pallas import tpu_sc as plsc

B = 4
S = 8192
D = 4096
E = 64

NC = 2
NS = 16
NW = NC * NS
CH = 8
NLANE = 16
UNROLL = 4

F_TC = 7680
T_SC = S - F_TC
BLKR = 512
NB_TC = F_TC // BLKR
ROWS_W = (B * T_SC) // NW


def _tc_sum_body(x_ref, o_ref, acc_ref):
    i = pl.program_id(0)
    nsteps = pl.num_programs(0)

    @pl.when(i == 0)
    def _init():
        acc_ref[...] = jnp.zeros_like(acc_ref)

    partial = jnp.sum(x_ref[...], axis=0, keepdims=True)
    bidx = i // NB_TC
    onehot = jax.lax.broadcasted_iota(jnp.int32, (B, 1), 0) == bidx
    acc_ref[...] += jnp.where(onehot, partial, 0.0)

    @pl.when(i == nsteps - 1)
    def _finish():
        o_ref[...] = acc_ref[...]


def _tc_sum(xf):
    grid = (B * NB_TC,)
    return pl.pallas_call(
        _tc_sum_body,
        grid=grid,
        in_specs=[
            pl.BlockSpec((BLKR, D),
                         lambda i: (i // NB_TC * (S // BLKR) + i % NB_TC, 0)),
        ],
        out_specs=pl.BlockSpec((B, D), lambda i: (0, 0)),
        out_shape=jax.ShapeDtypeStruct((B, D), jnp.float32),
        scratch_shapes=[pltpu.VMEM((B, D), jnp.float32)],
    )(xf)


def _sc_sum_body(x_hbm, out_hbm, buf, acc, sem0, sem1):
    wid = lax.axis_index("s") * NC + lax.axis_index("c")
    nch = ROWS_W // CH
    batch = wid // (NW // B)
    sub = wid % (NW // B)
    base = batch * S + F_TC + sub * ROWS_W

    def _zero(k, _):
        acc[pl.ds(k * NLANE, NLANE)] = jnp.zeros((NLANE,), jnp.float32)
        return 0

    lax.fori_loop(0, D // NLANE, _zero, 0)

    def _copy(c, slot, sem):
        return pltpu.make_async_copy(
            x_hbm.at[pl.ds(base + c * CH, CH), :], buf.at[slot], sem)

    _copy(0, 0, sem0).start()
    _copy(1, 1, sem1).start()

    def _chunk(c, slot, sem):
        _copy(c, slot, sem).wait()

        def _acc_slices(k, _):
            for u in range(UNROLL):
                o = (k * UNROLL + u) * NLANE
                t0 = buf[slot, 0, pl.ds(o, NLANE)] + buf[slot, 1, pl.ds(o, NLANE)]
                t1 = buf[slot, 2, pl.ds(o, NLANE)] + buf[slot, 3, pl.ds(o, NLANE)]
                t2 = buf[slot, 4, pl.ds(o, NLANE)] + buf[slot, 5, pl.ds(o, NLANE)]
                t3 = buf[slot, 6, pl.ds(o, NLANE)] + buf[slot, 7, pl.ds(o, NLANE)]
                acc[pl.ds(o, NLANE)] += (t0 + t1) + (t2 + t3)
            return 0

        lax.fori_loop(0, D // NLANE // UNROLL, _acc_slices, 0)

        @pl.when(c + 2 < nch)
        def _next():
            _copy(c + 2, slot, sem).start()

    def _pair(i, _):
        c0 = i * 2
        _chunk(c0, 0, sem0)
        _chunk(c0 + 1, 1, sem1)
        return 0

    lax.fori_loop(0, nch // 2, _pair, 0)

    pltpu.sync_copy(acc, out_hbm.at[wid])


def _sc_sum(xf):
    mesh = plsc.VectorSubcoreMesh(core_axis_name="c", subcore_axis_name="s")
    return pl.kernel(
        _sc_sum_body,
        out_type=jax.ShapeDtypeStruct((NW, D), jnp.float32),
        mesh=mesh,
        scratch_types=[
            pltpu.VMEM((2, CH, D), jnp.float32),
            pltpu.VMEM((D,), jnp.float32),
            pltpu.SemaphoreType.DMA,
            pltpu.SemaphoreType.DMA,
        ],
    )(xf)


def _finish_body(tc_ref, p_ref, w_ref, b_ref, o_ref):
    sel = (jax.lax.broadcasted_iota(jnp.int32, (B, NW), 1) // (NW // B)
           == jax.lax.broadcasted_iota(jnp.int32, (B, NW), 0)).astype(jnp.float32)
    total = tc_ref[...] + jax.lax.dot_general(
        sel, p_ref[...],
        dimension_numbers=(((1,), (0,)), ((), ())),
        preferred_element_type=jnp.float32,
    )
    xbar = total * (1.0 / S)
    scores = jax.lax.dot_general(
        xbar, w_ref[...],
        dimension_numbers=(((1,), (1,)), ((), ())),
        preferred_element_type=jnp.float32,
    ) + b_ref[...]
    m = jnp.max(scores, axis=1, keepdims=True)
    ex = jnp.exp(scores - m)
    o_ref[...] = ex / jnp.sum(ex, axis=1, keepdims=True)


def _finish(tc_partial, sc_partials, W, b2):
    return pl.pallas_call(
        _finish_body,
        in_specs=[
            pl.BlockSpec((B, D), lambda: (0, 0)),
            pl.BlockSpec((NW, D), lambda: (0, 0)),
            pl.BlockSpec((E, D), lambda: (0, 0)),
            pl.BlockSpec((1, E), lambda: (0, 0)),
        ],
        out_specs=pl.BlockSpec((B, E), lambda: (0, 0)),
        out_shape=jax.ShapeDtypeStruct((B, E), jnp.float32),
    )(tc_partial, sc_partials, W, b2)


def kernel(x, W, b):
    xf = x.reshape(B * S, D)
    sc_partials = _sc_sum(xf)
    tc_partial = _tc_sum(xf)
    return _finish(tc_partial, sc_partials, W, b.reshape(1, E))


# speedup vs baseline: 2.7343x; 1.1259x over previous
import jax
import jax.numpy as jnp
from jax.experimental import pallas as pl
from jax.experimental.pallas import tpu as pltpu

B = 4
S = 8192
D = 4096
E = 64

BLKR = 1024


def _body(x_ref, w_ref, b_ref, o_ref, acc_ref):
    i = pl.program_id(0)
    nsteps = pl.num_programs(0)

    @pl.when(i == 0)
    def _init():
        acc_ref[...] = jnp.zeros_like(acc_ref)

    partial = jnp.sum(x_ref[...], axis=0, keepdims=True)
    bidx = i // (S // BLKR)
    onehot = jax.lax.broadcasted_iota(jnp.int32, (B, 1), 0) == bidx
    acc_ref[...] += jnp.where(onehot, partial, 0.0)

    @pl.when(i == nsteps - 1)
    def _finish():
        xbar = acc_ref[...] * (1.0 / S)
        scores = jax.lax.dot_general(
            xbar, w_ref[...],
            dimension_numbers=(((1,), (1,)), ((), ())),
            preferred_element_type=jnp.float32,
        ) + b_ref[...]
        m = jnp.max(scores, axis=1, keepdims=True)
        ex = jnp.exp(scores - m)
        o_ref[...] = ex / jnp.sum(ex, axis=1, keepdims=True)


def kernel(x, W, b):
    xf = x.reshape(B * S, D)
    b2 = b.reshape(1, E)
    grid = (B * S // BLKR,)
    return pl.pallas_call(
        _body,
        grid=grid,
        in_specs=[
            pl.BlockSpec((BLKR, D), lambda i: (i, 0)),
            pl.BlockSpec((E, D), lambda i: (0, 0)),
            pl.BlockSpec((1, E), lambda i: (0, 0)),
        ],
        out_specs=pl.BlockSpec((B, E), lambda i: (0, 0)),
        out_shape=jax.ShapeDtypeStruct((B, E), jnp.float32),
        scratch_shapes=[pltpu.VMEM((B, D), jnp.float32)],
    )(xf, W, b2)
